# baseline (speedup 1.0000x reference)
import functools

import jax
import jax.numpy as jnp
from jax import lax
from jax.experimental import pallas as pl
from jax.experimental.pallas import tpu as pltpu
from jax.experimental.pallas import tpu_sc as plsc

NC = 2
NS = 16
NW = NC * NS

EB = 100
D = 128

_F32 = jnp.float32


def _sc_degree(dst3, n):
    nb = dst3.shape[1]
    assert n % 1000 == 0

    mesh = plsc.VectorSubcoreMesh(
        core_axis_name="c", subcore_axis_name="s", num_cores=NC)

    @functools.partial(
        pl.kernel,
        out_type=jax.ShapeDtypeStruct((NC, n), _F32),
        mesh=mesh,
        compiler_params=pltpu.CompilerParams(use_tc_tiling_on_sc=False),
        scratch_types=[
            pltpu.VMEM((nb, EB), jnp.int32),
            pltpu.VMEM((128,), _F32),
            pltpu.VMEM((1000,), _F32),
            pltpu.VMEM_SHARED((n,), _F32),
        ],
    )
    def deg_kernel(dst_hbm, out_hbm, dst_v, ones_v, zb_v, acc):
        cid = lax.axis_index("c")
        sid = lax.axis_index("s")
        wid = cid * NS + sid

        for i in range(8):
            ones_v[pl.ds(16 * i, 16)] = jnp.ones((16,), _F32)

        def zfill(i, carry):
            zb_v[pl.ds(16 * i, 16)] = jnp.zeros((16,), _F32)
            return carry
        lax.fori_loop(0, 62, zfill, 0)
        zb_v[pl.ds(984, 16)] = jnp.zeros((16,), _F32)

        nzt = n // 1000

        @pl.when(sid < nzt)
        def _zero():
            pltpu.sync_copy(zb_v, acc.at[pl.ds(sid * 1000, 1000)])

        plsc.subcore_barrier()

        pltpu.sync_copy(dst_hbm.at[wid], dst_v)

        def body(j, carry):
            pltpu.sync_copy(ones_v.at[pl.ds(0, EB)],
                            acc.at[dst_v.at[j]], add=True)
            return carry
        lax.fori_loop(0, nb, body, 0)

        plsc.subcore_barrier()

        @pl.when(sid < nzt)
        def _writeback():
            pltpu.sync_copy(acc.at[pl.ds(sid * 1000, 1000)],
                            out_hbm.at[cid, pl.ds(sid * 1000, 1000)])

    return deg_kernel(dst3)


def _sc_aggregate(src3, dst3, z):
    nb = src3.shape[1]
    n = z.shape[0]
    assert n % NS == 0

    mesh = plsc.VectorSubcoreMesh(
        core_axis_name="c", subcore_axis_name="s", num_cores=NC)

    @functools.partial(
        pl.kernel,
        out_type=jax.ShapeDtypeStruct((NC, n, D), _F32),
        mesh=mesh,
        compiler_params=pltpu.CompilerParams(use_tc_tiling_on_sc=False),
        scratch_types=[
            pltpu.VMEM((nb, EB), jnp.int32),
            pltpu.VMEM((nb, EB), jnp.int32),
            pltpu.VMEM((EB, D), _F32),
            pltpu.VMEM((EB, D), _F32),
            pltpu.VMEM((EB, D), _F32),
            pltpu.VMEM_SHARED((n, D), _F32),
            pltpu.SemaphoreType.DMA,
            pltpu.SemaphoreType.DMA,
            pltpu.SemaphoreType.DMA,
            pltpu.SemaphoreType.DMA,
            pltpu.SemaphoreType.DMA,
            pltpu.SemaphoreType.DMA,
            pltpu.SemaphoreType.DMA,
            pltpu.SemaphoreType.DMA,
        ],
    )
    def agg_kernel(src_hbm, dst_hbm, z_hbm, zeros_hbm, out_hbm,
                   src_v, dst_v, rows0_v, rows1_v, rows2_v, acc,
                   isem, zsem, gsem0, gsem1, gsem2, ssem0, ssem1, ssem2):
        cid = lax.axis_index("c")
        sid = lax.axis_index("s")
        wid = cid * NS + sid
        rows = (rows0_v, rows1_v, rows2_v)
        gsem = (gsem0, gsem1, gsem2)
        ssem = (ssem0, ssem1, ssem2)

        id0 = pltpu.async_copy(src_hbm.at[wid], src_v, isem)
        id1 = pltpu.async_copy(dst_hbm.at[wid], dst_v, isem)

        npt = n // NS
        pltpu.async_copy(zeros_hbm, acc.at[pl.ds(sid * npt, npt)],
                         zsem).wait()

        id0.wait()
        id1.wait()

        def gather(j, b):
            return pltpu.async_copy(z_hbm.at[src_v.at[j]], rows[b], gsem[b])

        def scatter(j, b):
            return pltpu.async_copy(rows[b], acc.at[dst_v.at[j]], ssem[b],
                                    add=True)

        gd = {j: gather(j, j) for j in range(min(3, nb))}

        plsc.subcore_barrier()

        sd = {}
        for j in range(nb):
            b = j % 3
            gd[j].wait()
            if j >= 1:
                sd[j - 1].wait()
                if j + 2 < nb:
                    gd[j + 2] = gather(j + 2, (j + 2) % 3)
            sd[j] = scatter(j, b)
        sd[nb - 1].wait()

        plsc.subcore_barrier()

        pltpu.sync_copy(acc.at[pl.ds(sid * npt, npt)],
                        out_hbm.at[cid, pl.ds(sid * npt, npt)])

    zeros = jnp.zeros((n // NS, D), _F32)
    return agg_kernel(src3, dst3, z, zeros)


_RB = 1000


def _tc_matmul(x, w0):
    n, din = x.shape
    dout = w0.shape[1]

    def body(x_ref, w_ref, o_ref):
        o_ref[...] = jnp.dot(x_ref[...], w_ref[...],
                             preferred_element_type=_F32)

    return pl.pallas_call(
        body,
        grid=(n // _RB,),
        in_specs=[
            pl.BlockSpec((_RB, din), lambda i: (i, 0)),
            pl.BlockSpec((din, dout), lambda i: (0, 0)),
        ],
        out_specs=pl.BlockSpec((_RB, dout), lambda i: (i, 0)),
        out_shape=jax.ShapeDtypeStruct((n, dout), _F32),
    )(x, w0)


def _tc_scale(xw, dinvb):
    n, d = xw.shape

    def body(x_ref, dinv_ref, o_ref):
        o_ref[...] = x_ref[...] * dinv_ref[...]

    return pl.pallas_call(
        body,
        grid=(n // _RB,),
        in_specs=[
            pl.BlockSpec((_RB, d), lambda i: (i, 0)),
            pl.BlockSpec((_RB, d), lambda i: (i, 0)),
        ],
        out_specs=pl.BlockSpec((_RB, d), lambda i: (i, 0)),
        out_shape=jax.ShapeDtypeStruct((n, d), _F32),
    )(xw, dinvb)


def _tc_layer(s, z, dinvb, b, w, split_out=False):
    n, d = z.shape
    dout = w.shape[1]
    b2 = b.reshape(1, d)

    def body(s_ref, z_ref, dinv_ref, b_ref, w_ref, *o_refs):
        agg = s_ref[0] + s_ref[1] + z_ref[...]
        t = dinv_ref[...] * agg + b_ref[...]
        h = jnp.where(t > 0, t, 0.01 * t)
        zn = jnp.dot(dinv_ref[...] * h, w_ref[...],
                     preferred_element_type=_F32)
        if split_out:
            o_refs[0][...] = zn[:, :dout // 2]
            o_refs[1][...] = zn[:, dout // 2:]
        else:
            o_refs[0][...] = zn

    if split_out:
        out_specs = [pl.BlockSpec((_RB, dout // 2), lambda i: (i, 0)),
                     pl.BlockSpec((_RB, dout // 2), lambda i: (i, 0))]
        out_shape = [jax.ShapeDtypeStruct((n, dout // 2), _F32),
                     jax.ShapeDtypeStruct((n, dout // 2), _F32)]
    else:
        out_specs = pl.BlockSpec((_RB, dout), lambda i: (i, 0))
        out_shape = jax.ShapeDtypeStruct((n, dout), _F32)

    return pl.pallas_call(
        body,
        grid=(n // _RB,),
        in_specs=[
            pl.BlockSpec((NC, _RB, d), lambda i: (0, i, 0)),
            pl.BlockSpec((_RB, d), lambda i: (i, 0)),
            pl.BlockSpec((_RB, d), lambda i: (i, 0)),
            pl.BlockSpec((1, d), lambda i: (0, 0)),
            pl.BlockSpec((d, dout), lambda i: (0, 0)),
        ],
        out_specs=out_specs,
        out_shape=out_shape,
    )(s, z, dinvb, b2, w)


def _tc_head(sa, sb, z7a, z7b, dinvb, b7, wl1, bl1, wl2, bl2):
    n, d = z7a.shape
    d2 = 2 * d
    b7r = b7.reshape(1, d2)
    bl1r = bl1.reshape(1, d2)
    bl2r = bl2.reshape(1, d2)

    def body(sa_ref, sb_ref, za_ref, zb_ref, dinv_ref, b7_ref, wl1_ref,
             bl1_ref, wl2_ref, bl2_ref, o_ref):
        agg = jnp.concatenate(
            [sa_ref[0] + sa_ref[1] + za_ref[...],
             sb_ref[0] + sb_ref[1] + zb_ref[...]], axis=1)
        t = dinv_ref[...] * agg + b7_ref[...]
        h = jnp.where(t > 0, t, 0.01 * t)
        u = jnp.dot(h, wl1_ref[...], preferred_element_type=_F32) + bl1_ref[...]
        u = jnp.maximum(u, 0.0)
        o_ref[...] = (jnp.dot(u, wl2_ref[...], preferred_element_type=_F32)
                      + bl2_ref[...])

    return pl.pallas_call(
        body,
        grid=(n // _RB,),
        in_specs=[
            pl.BlockSpec((NC, _RB, d), lambda i: (0, i, 0)),
            pl.BlockSpec((NC, _RB, d), lambda i: (0, i, 0)),
            pl.BlockSpec((_RB, d), lambda i: (i, 0)),
            pl.BlockSpec((_RB, d), lambda i: (i, 0)),
            pl.BlockSpec((_RB, d2), lambda i: (i, 0)),
            pl.BlockSpec((1, d2), lambda i: (0, 0)),
            pl.BlockSpec((d2, d2), lambda i: (0, 0)),
            pl.BlockSpec((1, d2), lambda i: (0, 0)),
            pl.BlockSpec((d2, d2), lambda i: (0, 0)),
            pl.BlockSpec((1, d2), lambda i: (0, 0)),
        ],
        out_specs=pl.BlockSpec((_RB, d2), lambda i: (i, 0)),
        out_shape=jax.ShapeDtypeStruct((n, d2), _F32),
    )(sa, sb, z7a, z7b, dinvb, b7r, wl1, bl1r, wl2, bl2r)


def kernel(x, edge_index, W0, b0, W1, b1, W2, b2, W3, b3, W4, b4, W5, b5,
           W6, b6, W7, b7, Wl1, bl1, Wl2, bl2):
    n, dx = x.shape
    e = edge_index.shape[1]
    assert e % (NW * EB) == 0, "edge count must tile across 32 subcores"
    nb = e // (NW * EB)

    src3 = edge_index[0].reshape(NW, nb, EB)
    dst3 = edge_index[1].reshape(NW, nb, EB)

    xw0 = _tc_matmul(x, W0)
    degp = _sc_degree(dst3, n)
    deg = degp[0] + degp[1] + 1.0
    dinv = lax.rsqrt(deg)
    dinvb = jnp.broadcast_to(dinv[:, None], (n, dx))

    z = _tc_scale(xw0, dinvb[:, :xw0.shape[1]])
    layer_bw = [(b0, W1), (b1, W2), (b2, W3), (b3, W4), (b4, W5), (b5, W6)]
    for b, w in layer_bw:
        s = _sc_aggregate(src3, dst3, z)
        z = _tc_layer(s, z, dinvb, b, w)

    s = _sc_aggregate(src3, dst3, z)
    z7a, z7b = _tc_layer(s, z, dinvb, b6, W7, split_out=True)
    sa = _sc_aggregate(src3, dst3, z7a)
    sb = _sc_aggregate(src3, dst3, z7b)
    return _tc_head(sa, sb, z7a, z7b, dinvb, b7, Wl1, bl1, Wl2, bl2)


# speedup vs baseline: 1.0027x; 1.0027x over previous
import functools

import jax
import jax.numpy as jnp
from jax import lax
from jax.experimental import pallas as pl
from jax.experimental.pallas import tpu as pltpu
from jax.experimental.pallas import tpu_sc as plsc

NC = 2
NS = 16
NW = NC * NS

EB = 100
D = 128

_F32 = jnp.float32


def _sc_degree(dst3, n):
    nb = dst3.shape[1]
    assert n % 1000 == 0

    mesh = plsc.VectorSubcoreMesh(
        core_axis_name="c", subcore_axis_name="s", num_cores=NC)

    @functools.partial(
        pl.kernel,
        out_type=jax.ShapeDtypeStruct((NC, n), _F32),
        mesh=mesh,
        compiler_params=pltpu.CompilerParams(use_tc_tiling_on_sc=False),
        scratch_types=[
            pltpu.VMEM((nb, EB), jnp.int32),
            pltpu.VMEM((128,), _F32),
            pltpu.VMEM((1000,), _F32),
            pltpu.VMEM_SHARED((n,), _F32),
        ],
    )
    def deg_kernel(dst_hbm, out_hbm, dst_v, ones_v, zb_v, acc):
        cid = lax.axis_index("c")
        sid = lax.axis_index("s")
        wid = cid * NS + sid

        for i in range(8):
            ones_v[pl.ds(16 * i, 16)] = jnp.ones((16,), _F32)

        def zfill(i, carry):
            zb_v[pl.ds(16 * i, 16)] = jnp.zeros((16,), _F32)
            return carry
        lax.fori_loop(0, 62, zfill, 0)
        zb_v[pl.ds(984, 16)] = jnp.zeros((16,), _F32)

        nzt = n // 1000

        @pl.when(sid < nzt)
        def _zero():
            pltpu.sync_copy(zb_v, acc.at[pl.ds(sid * 1000, 1000)])

        plsc.subcore_barrier()

        pltpu.sync_copy(dst_hbm.at[wid], dst_v)

        def body(j, carry):
            pltpu.sync_copy(ones_v.at[pl.ds(0, EB)],
                            acc.at[dst_v.at[j]], add=True)
            return carry
        lax.fori_loop(0, nb, body, 0)

        plsc.subcore_barrier()

        @pl.when(sid < nzt)
        def _writeback():
            pltpu.sync_copy(acc.at[pl.ds(sid * 1000, 1000)],
                            out_hbm.at[cid, pl.ds(sid * 1000, 1000)])

    return deg_kernel(dst3)


def _sc_aggregate(src3, dst3, *zs):
    nb = src3.shape[1]
    n = zs[0].shape[0]
    np_ = len(zs)
    assert n % NS == 0

    mesh = plsc.VectorSubcoreMesh(
        core_axis_name="c", subcore_axis_name="s", num_cores=NC)

    @functools.partial(
        pl.kernel,
        out_type=tuple(jax.ShapeDtypeStruct((NC, n, D), _F32)
                       for _ in range(np_)),
        mesh=mesh,
        compiler_params=pltpu.CompilerParams(use_tc_tiling_on_sc=False),
        scratch_types=[
            pltpu.VMEM((nb, EB), jnp.int32),
            pltpu.VMEM((nb, EB), jnp.int32),
            pltpu.VMEM((EB, D), _F32),
            pltpu.VMEM((EB, D), _F32),
            pltpu.VMEM((EB, D), _F32),
            pltpu.VMEM_SHARED((n, D), _F32),
            pltpu.SemaphoreType.DMA,
            pltpu.SemaphoreType.DMA,
            pltpu.SemaphoreType.DMA,
            pltpu.SemaphoreType.DMA,
            pltpu.SemaphoreType.DMA,
            pltpu.SemaphoreType.DMA,
            pltpu.SemaphoreType.DMA,
            pltpu.SemaphoreType.DMA,
        ],
    )
    def agg_kernel(src_hbm, dst_hbm, *rest):
        z_hbms = rest[:np_]
        zeros_hbm = rest[np_]
        out_hbms = rest[np_ + 1:2 * np_ + 1]
        (src_v, dst_v, rows0_v, rows1_v, rows2_v, acc,
         isem, zsem, gsem0, gsem1, gsem2, ssem0, ssem1, ssem2) = (
            rest[2 * np_ + 1:])
        cid = lax.axis_index("c")
        sid = lax.axis_index("s")
        wid = cid * NS + sid
        rows = (rows0_v, rows1_v, rows2_v)
        gsem = (gsem0, gsem1, gsem2)
        ssem = (ssem0, ssem1, ssem2)
        npt = n // NS
        slab = pl.ds(sid * npt, npt)

        id0 = pltpu.async_copy(src_hbm.at[wid], src_v, isem)
        id1 = pltpu.async_copy(dst_hbm.at[wid], dst_v, isem)
        pltpu.async_copy(zeros_hbm, acc.at[slab], zsem).wait()
        id0.wait()
        id1.wait()

        def gather(z_hbm, j, b):
            return pltpu.async_copy(z_hbm.at[src_v.at[j]], rows[b], gsem[b])

        def scatter(j, b):
            return pltpu.async_copy(rows[b], acc.at[dst_v.at[j]], ssem[b],
                                    add=True)

        for p in range(np_):
            z_hbm = z_hbms[p]
            gd = {j: gather(z_hbm, j, j) for j in range(min(3, nb))}

            plsc.subcore_barrier()

            sd = {}
            for j in range(nb):
                b = j % 3
                gd[j].wait()
                if j >= 1:
                    sd[j - 1].wait()
                    if j + 2 < nb:
                        gd[j + 2] = gather(z_hbm, j + 2, (j + 2) % 3)
                sd[j] = scatter(j, b)
            sd[nb - 1].wait()

            plsc.subcore_barrier()

            pltpu.sync_copy(acc.at[slab], out_hbms[p].at[cid, slab])
            if p + 1 < np_:
                pltpu.sync_copy(zeros_hbm, acc.at[slab])

    zeros = jnp.zeros((n // NS, D), _F32)
    outs = agg_kernel(src3, dst3, *zs, zeros)
    return outs[0] if np_ == 1 else outs


_RB = 1000


def _tc_matmul(x, w0):
    n, din = x.shape
    dout = w0.shape[1]

    def body(x_ref, w_ref, o_ref):
        o_ref[...] = jnp.dot(x_ref[...], w_ref[...],
                             preferred_element_type=_F32)

    return pl.pallas_call(
        body,
        grid=(n // _RB,),
        in_specs=[
            pl.BlockSpec((_RB, din), lambda i: (i, 0)),
            pl.BlockSpec((din, dout), lambda i: (0, 0)),
        ],
        out_specs=pl.BlockSpec((_RB, dout), lambda i: (i, 0)),
        out_shape=jax.ShapeDtypeStruct((n, dout), _F32),
    )(x, w0)


def _tc_scale(xw, dinvb):
    n, d = xw.shape

    def body(x_ref, dinv_ref, o_ref):
        o_ref[...] = x_ref[...] * dinv_ref[...]

    return pl.pallas_call(
        body,
        grid=(n // _RB,),
        in_specs=[
            pl.BlockSpec((_RB, d), lambda i: (i, 0)),
            pl.BlockSpec((_RB, d), lambda i: (i, 0)),
        ],
        out_specs=pl.BlockSpec((_RB, d), lambda i: (i, 0)),
        out_shape=jax.ShapeDtypeStruct((n, d), _F32),
    )(xw, dinvb)


def _tc_layer(s, z, dinvb, b, w, split_out=False):
    n, d = z.shape
    dout = w.shape[1]
    b2 = b.reshape(1, d)

    def body(s_ref, z_ref, dinv_ref, b_ref, w_ref, *o_refs):
        agg = s_ref[0] + s_ref[1] + z_ref[...]
        t = dinv_ref[...] * agg + b_ref[...]
        h = jnp.where(t > 0, t, 0.01 * t)
        zn = jnp.dot(dinv_ref[...] * h, w_ref[...],
                     preferred_element_type=_F32)
        if split_out:
            o_refs[0][...] = zn[:, :dout // 2]
            o_refs[1][...] = zn[:, dout // 2:]
        else:
            o_refs[0][...] = zn

    if split_out:
        out_specs = [pl.BlockSpec((_RB, dout // 2), lambda i: (i, 0)),
                     pl.BlockSpec((_RB, dout // 2), lambda i: (i, 0))]
        out_shape = [jax.ShapeDtypeStruct((n, dout // 2), _F32),
                     jax.ShapeDtypeStruct((n, dout // 2), _F32)]
    else:
        out_specs = pl.BlockSpec((_RB, dout), lambda i: (i, 0))
        out_shape = jax.ShapeDtypeStruct((n, dout), _F32)

    return pl.pallas_call(
        body,
        grid=(n // _RB,),
        in_specs=[
            pl.BlockSpec((NC, _RB, d), lambda i: (0, i, 0)),
            pl.BlockSpec((_RB, d), lambda i: (i, 0)),
            pl.BlockSpec((_RB, d), lambda i: (i, 0)),
            pl.BlockSpec((1, d), lambda i: (0, 0)),
            pl.BlockSpec((d, dout), lambda i: (0, 0)),
        ],
        out_specs=out_specs,
        out_shape=out_shape,
    )(s, z, dinvb, b2, w)


def _tc_head(sa, sb, z7a, z7b, dinvb, b7, wl1, bl1, wl2, bl2):
    n, d = z7a.shape
    d2 = 2 * d
    b7r = b7.reshape(1, d2)
    bl1r = bl1.reshape(1, d2)
    bl2r = bl2.reshape(1, d2)

    def body(sa_ref, sb_ref, za_ref, zb_ref, dinv_ref, b7_ref, wl1_ref,
             bl1_ref, wl2_ref, bl2_ref, o_ref):
        agg = jnp.concatenate(
            [sa_ref[0] + sa_ref[1] + za_ref[...],
             sb_ref[0] + sb_ref[1] + zb_ref[...]], axis=1)
        t = dinv_ref[...] * agg + b7_ref[...]
        h = jnp.where(t > 0, t, 0.01 * t)
        u = jnp.dot(h, wl1_ref[...], preferred_element_type=_F32) + bl1_ref[...]
        u = jnp.maximum(u, 0.0)
        o_ref[...] = (jnp.dot(u, wl2_ref[...], preferred_element_type=_F32)
                      + bl2_ref[...])

    return pl.pallas_call(
        body,
        grid=(n // _RB,),
        in_specs=[
            pl.BlockSpec((NC, _RB, d), lambda i: (0, i, 0)),
            pl.BlockSpec((NC, _RB, d), lambda i: (0, i, 0)),
            pl.BlockSpec((_RB, d), lambda i: (i, 0)),
            pl.BlockSpec((_RB, d), lambda i: (i, 0)),
            pl.BlockSpec((_RB, d2), lambda i: (i, 0)),
            pl.BlockSpec((1, d2), lambda i: (0, 0)),
            pl.BlockSpec((d2, d2), lambda i: (0, 0)),
            pl.BlockSpec((1, d2), lambda i: (0, 0)),
            pl.BlockSpec((d2, d2), lambda i: (0, 0)),
            pl.BlockSpec((1, d2), lambda i: (0, 0)),
        ],
        out_specs=pl.BlockSpec((_RB, d2), lambda i: (i, 0)),
        out_shape=jax.ShapeDtypeStruct((n, d2), _F32),
    )(sa, sb, z7a, z7b, dinvb, b7r, wl1, bl1r, wl2, bl2r)


def kernel(x, edge_index, W0, b0, W1, b1, W2, b2, W3, b3, W4, b4, W5, b5,
           W6, b6, W7, b7, Wl1, bl1, Wl2, bl2):
    n, dx = x.shape
    e = edge_index.shape[1]
    assert e % (NW * EB) == 0, "edge count must tile across 32 subcores"
    nb = e // (NW * EB)

    src3 = edge_index[0].reshape(NW, nb, EB)
    dst3 = edge_index[1].reshape(NW, nb, EB)

    xw0 = _tc_matmul(x, W0)
    degp = _sc_degree(dst3, n)
    deg = degp[0] + degp[1] + 1.0
    dinv = lax.rsqrt(deg)
    dinvb = jnp.broadcast_to(dinv[:, None], (n, dx))

    z = _tc_scale(xw0, dinvb[:, :xw0.shape[1]])
    layer_bw = [(b0, W1), (b1, W2), (b2, W3), (b3, W4), (b4, W5), (b5, W6)]
    for b, w in layer_bw:
        s = _sc_aggregate(src3, dst3, z)
        z = _tc_layer(s, z, dinvb, b, w)

    s = _sc_aggregate(src3, dst3, z)
    z7a, z7b = _tc_layer(s, z, dinvb, b6, W7, split_out=True)
    sa, sb = _sc_aggregate(src3, dst3, z7a, z7b)
    return _tc_head(sa, sb, z7a, z7b, dinvb, b7, Wl1, bl1, Wl2, bl2)
